# Initial kernel scaffold; baseline (speedup 1.0000x reference)
import jax
import jax.numpy as jnp
from jax.experimental import pallas as pl


def kernel(x, table):
    raise NotImplementedError("write your pallas kernel here")


# speedup vs baseline: 1.0299x; 1.0299x over previous
import functools

import numpy as np
import jax
import jax.numpy as jnp
from jax import lax
from jax.experimental import pallas as pl
from jax.experimental.pallas import tpu as pltpu
from jax.experimental.pallas import tpu_sc as plsc


def _pos_encoding_np(length: int, depth: int) -> np.ndarray:
    half = depth / 2
    positions = np.arange(length)[:, np.newaxis]
    depths = np.arange(half)[np.newaxis, :] / half
    angle_rates = 1 / 10000 ** depths
    angle_rads = positions * angle_rates
    return np.concatenate(
        [np.sin(angle_rads), np.cos(angle_rads)], axis=-1
    ).astype(np.float32)


def kernel(x, table):
    B, L = x.shape
    V, D = table.shape
    N = B * L
    scale = float(np.sqrt(float(D)))

    info = plsc.get_sparse_core_info()
    NC, NS, LN = info.num_cores, info.num_subcores, info.num_lanes
    NW = NC * NS
    b_per_w = N // NW
    GRP = 128
    n_grp = b_per_w // GRP

    pos = jnp.asarray(_pos_encoding_np(L, D))
    xf = x.reshape(N // GRP, GRP)

    mesh = plsc.VectorSubcoreMesh(core_axis_name="c", subcore_axis_name="s")

    @functools.partial(
        pl.kernel,
        mesh=mesh,
        out_type=jax.ShapeDtypeStruct((N, D), jnp.float32),
        scratch_types=[
            pltpu.VMEM((n_grp, GRP), jnp.int32),
            pltpu.VMEM((b_per_w, D), jnp.float32),
            pltpu.VMEM((b_per_w, D), jnp.float32),
            pltpu.SemaphoreType.DMA,
        ],
    )
    def emb_kernel(x_hbm, tab_hbm, pos_hbm, out_hbm, idx_v, rows_v, pos_v, sem):
        wid = lax.axis_index("s") * NC + lax.axis_index("c")
        base = wid * b_per_w
        pltpu.sync_copy(x_hbm.at[pl.ds(wid * n_grp, n_grp)], idx_v)
        poff = lax.rem(base, L)
        pltpu.sync_copy(pos_hbm.at[pl.ds(poff, b_per_w)], pos_v)
        copies = [
            pltpu.async_copy(
                tab_hbm.at[idx_v.at[j]], rows_v.at[pl.ds(j * GRP, GRP)], sem
            )
            for j in range(n_grp)
        ]
        for c in copies:
            c.wait()

        def body(i, carry):
            for k in range(D // LN):
                sl = pl.ds(k * LN, LN)
                rows_v[i, sl] = rows_v[i, sl] * scale + pos_v[i, sl]
            return carry

        lax.fori_loop(0, b_per_w, body, 0)
        pltpu.sync_copy(rows_v, out_hbm.at[pl.ds(base, b_per_w)])

    out = emb_kernel(xf, table, pos)
    return out.reshape(B, L, D)


# speedup vs baseline: 1.1374x; 1.1043x over previous
import functools

import numpy as np
import jax
import jax.numpy as jnp
from jax import lax
from jax.experimental import pallas as pl
from jax.experimental.pallas import tpu as pltpu
from jax.experimental.pallas import tpu_sc as plsc


def _pos_encoding_np(length: int, depth: int) -> np.ndarray:
    half = depth / 2
    positions = np.arange(length)[:, np.newaxis]
    depths = np.arange(half)[np.newaxis, :] / half
    angle_rates = 1 / 10000 ** depths
    angle_rads = positions * angle_rates
    return np.concatenate(
        [np.sin(angle_rads), np.cos(angle_rads)], axis=-1
    ).astype(np.float32)


def kernel(x, table):
    B, L = x.shape
    V, D = table.shape
    N = B * L
    scale = float(np.sqrt(float(D)))

    info = plsc.get_sparse_core_info()
    NC, NS, LN = info.num_cores, info.num_subcores, info.num_lanes
    NW = NC * NS
    P = L // NW
    VPR = D // LN

    pos = jnp.asarray(_pos_encoding_np(L, D))
    xr = x.reshape(B * NW, P)

    mesh = plsc.VectorSubcoreMesh(core_axis_name="c", subcore_axis_name="s")

    @functools.partial(
        pl.kernel,
        mesh=mesh,
        out_type=jax.ShapeDtypeStruct((N, D), jnp.float32),
        scratch_types=[
            pltpu.VMEM((B, P), jnp.int32),
            pltpu.VMEM((B * P, D), jnp.float32),
            pltpu.VMEM((P, D), jnp.float32),
            pltpu.SemaphoreType.DMA,
            pltpu.SemaphoreType.DMA,
            pltpu.SemaphoreType.DMA,
        ],
    )
    def emb_kernel(
        x_hbm, tab_hbm, pos_hbm, out_hbm, idx_v, rows_v, pos_v, gsem, osem, isem
    ):
        wid = lax.axis_index("s") * NC + lax.axis_index("c")
        icopies = [
            pltpu.async_copy(
                x_hbm.at[pl.ds(b * NW + wid, 1)], idx_v.at[pl.ds(b, 1)], isem
            )
            for b in range(B)
        ]
        for c in icopies:
            c.wait()
        gathers = [None] * B
        gathers[0] = pltpu.async_copy(
            tab_hbm.at[idx_v.at[0]], rows_v.at[pl.ds(0, P)], gsem
        )
        pltpu.sync_copy(pos_hbm.at[pl.ds(wid * P, P)], pos_v)

        wb = []
        for b in range(B):
            if b + 1 < B:
                gathers[b + 1] = pltpu.async_copy(
                    tab_hbm.at[idx_v.at[b + 1]],
                    rows_v.at[pl.ds((b + 1) * P, P)],
                    gsem,
                )
            gathers[b].wait()

            def body(i, carry, b=b):
                r = b * P + i
                for k in range(VPR):
                    sl = pl.ds(k * LN, LN)
                    rows_v[r, sl] = rows_v[r, sl] * scale + pos_v[i, sl]
                return carry

            lax.fori_loop(0, P, body, 0)
            wb.append(
                pltpu.async_copy(
                    rows_v.at[pl.ds(b * P, P)],
                    out_hbm.at[pl.ds(b * L + wid * P, P)],
                    osem,
                )
            )
        for c in wb:
            c.wait()

    out = emb_kernel(xr, table, pos)
    return out.reshape(B, L, D)
